# baseline (speedup 1.0000x reference)
import jax
import jax.numpy as jnp
from jax.experimental import pallas as pl

N_DIRS = 16384
D_MODEL = 2048
K = 32
MULTIK = 128


def _bias_kernel(x_ref, b_ref, o_ref):
    o_ref[...] = x_ref[...] - b_ref[...]


def kernel(x, W_enc, W_dec, pre_bias, latent_bias):
    xc = pl.pallas_call(
        _bias_kernel,
        grid=(x.shape[0] // 512,),
        in_specs=[
            pl.BlockSpec((512, x.shape[1]), lambda i: (i, 0)),
            pl.BlockSpec((512, x.shape[1]), lambda i: (i, 0)),
        ],
        out_specs=pl.BlockSpec((512, x.shape[1]), lambda i: (i, 0)),
        out_shape=jax.ShapeDtypeStruct(x.shape, x.dtype),
    )(x, jnp.broadcast_to(pre_bias, x.shape))
    latents_pre_act = xc @ W_enc.T + latent_bias
    topk_values, topk_indices = jax.lax.top_k(latents_pre_act, K)
    topk_values = jax.nn.relu(topk_values)
    multik_values, multik_indices = jax.lax.top_k(latents_pre_act, MULTIK)
    multik_values = jax.nn.relu(multik_values)
    rows = jnp.arange(latents_pre_act.shape[0])[:, None]
    latents = jnp.zeros_like(latents_pre_act).at[rows, topk_indices].set(topk_values)
    multik_latents = jnp.zeros_like(latents_pre_act).at[rows, multik_indices].set(multik_values)
    recons = latents @ W_dec.T + pre_bias
    multik_recons = multik_latents @ W_dec.T + pre_bias
    return (recons, topk_indices, topk_values, multik_indices, multik_values,
            multik_recons, latents_pre_act, latents)


# speedup vs baseline: 1.1458x; 1.1458x over previous
import functools

import jax
import jax.numpy as jnp
from jax import lax
from jax.experimental import pallas as pl
from jax.experimental.pallas import tpu as pltpu

N_DIRS = 16384
D_MODEL = 2048
K = 32
MULTIK = 128

ROWS = 8192
RB_ENC = 512
CB = 2048
NJ = N_DIRS // CB
NI_ENC = ROWS // RB_ENC
RB_DEC = 512
NI_DEC = ROWS // RB_DEC
CBD = 1024
NJD = N_DIRS // CBD


def _enc_kernel(x_ref, wenc_ref, lb_ref, pb_ref, out_ref, s1_ref, s2_ref):
    xc = x_ref[...] - pb_ref[...]
    acc = jax.lax.dot_general(
        xc, wenc_ref[...],
        dimension_numbers=(((1,), (1,)), ((), ())),
        preferred_element_type=jnp.float32,
    )
    acc = acc + lb_ref[...]
    out_ref[...] = acc
    s1_ref[...] = jnp.sum(acc, axis=1)[None, None, :]
    s2_ref[...] = jnp.sum(acc * acc, axis=1)[None, None, :]


def _encoder(x, W_enc, pre_bias, latent_bias):
    grid = (NJ, NI_ENC)
    out_shapes = (
        jax.ShapeDtypeStruct((ROWS, N_DIRS), jnp.float32),
        jax.ShapeDtypeStruct((NJ, 1, ROWS), jnp.float32),
        jax.ShapeDtypeStruct((NJ, 1, ROWS), jnp.float32),
    )
    return pl.pallas_call(
        _enc_kernel,
        grid=grid,
        in_specs=[
            pl.BlockSpec((RB_ENC, D_MODEL), lambda j, i: (i, 0)),
            pl.BlockSpec((CB, D_MODEL), lambda j, i: (j, 0)),
            pl.BlockSpec((1, CB), lambda j, i: (0, j)),
            pl.BlockSpec((1, D_MODEL), lambda j, i: (0, 0)),
        ],
        out_specs=(
            pl.BlockSpec((RB_ENC, CB), lambda j, i: (i, j)),
            pl.BlockSpec((1, 1, RB_ENC), lambda j, i: (j, 0, i)),
            pl.BlockSpec((1, 1, RB_ENC), lambda j, i: (j, 0, i)),
        ),
        out_shape=out_shapes,
    )(x, W_enc, latent_bias.reshape(1, N_DIRS), pre_bias.reshape(1, D_MODEL))


def _dec_kernel(pre_ref, t32_ref, t128_ref, wdec_ref, pb_ref,
                lat_ref, rec_ref, mrec_ref):
    j = pl.program_id(1)
    pre = pre_ref[...]
    relu = jnp.maximum(pre, 0.0)
    lat = jnp.where(pre >= t32_ref[...], relu, 0.0)
    mk = jnp.where(pre >= t128_ref[...], relu, 0.0)
    lat_ref[...] = lat
    w = wdec_ref[...]
    prec = jax.lax.Precision.DEFAULT
    rec = jax.lax.dot_general(
        lat, w, dimension_numbers=(((1,), (1,)), ((), ())),
        preferred_element_type=jnp.float32, precision=prec)
    mrec = jax.lax.dot_general(
        mk, w, dimension_numbers=(((1,), (1,)), ((), ())),
        preferred_element_type=jnp.float32, precision=prec)

    @pl.when(j == 0)
    def _init():
        rec_ref[...] = rec + pb_ref[...]
        mrec_ref[...] = mrec + pb_ref[...]

    @pl.when(j > 0)
    def _acc():
        rec_ref[...] += rec
        mrec_ref[...] += mrec


def _decoder(pre, tau32, tau128, W_dec, pre_bias):
    grid = (NI_DEC, NJD)
    out_shapes = (
        jax.ShapeDtypeStruct((ROWS, N_DIRS), jnp.float32),
        jax.ShapeDtypeStruct((ROWS, D_MODEL), jnp.float32),
        jax.ShapeDtypeStruct((ROWS, D_MODEL), jnp.float32),
    )
    return pl.pallas_call(
        _dec_kernel,
        grid=grid,
        in_specs=[
            pl.BlockSpec((RB_DEC, CBD), lambda i, j: (i, j)),
            pl.BlockSpec((RB_DEC, 1), lambda i, j: (i, 0)),
            pl.BlockSpec((RB_DEC, 1), lambda i, j: (i, 0)),
            pl.BlockSpec((D_MODEL, CBD), lambda i, j: (0, j)),
            pl.BlockSpec((1, D_MODEL), lambda i, j: (0, 0)),
        ],
        out_specs=(
            pl.BlockSpec((RB_DEC, CBD), lambda i, j: (i, j)),
            pl.BlockSpec((RB_DEC, D_MODEL), lambda i, j: (i, 0)),
            pl.BlockSpec((RB_DEC, D_MODEL), lambda i, j: (i, 0)),
        ),
        out_shape=out_shapes,
    )(pre, tau32, tau128, W_dec, pre_bias.reshape(1, D_MODEL))


def kernel(x, W_enc, W_dec, pre_bias, latent_bias):
    pre, s1, s2 = _encoder(x, W_enc, pre_bias, latent_bias)

    multik_values_raw, multik_indices = jax.lax.top_k(pre, MULTIK)
    topk_indices = multik_indices[:, :K]
    tau32 = multik_values_raw[:, K - 1:K]
    tau128 = multik_values_raw[:, MULTIK - 1:MULTIK]

    multik_values = jnp.maximum(multik_values_raw, 0.0)
    topk_values = multik_values[:, :K]

    latents, recons, multik_recons = _decoder(pre, tau32, tau128, W_dec, pre_bias)
    return (recons, topk_indices, topk_values, multik_indices, multik_values,
            multik_recons, pre, latents)


# speedup vs baseline: 7.8064x; 6.8130x over previous
import functools

import numpy as _np
import jax
import jax.numpy as jnp
from jax import lax
from jax.experimental import pallas as pl
from jax.experimental.pallas import tpu as pltpu
from jax.experimental.pallas import tpu_sc as plsc

N_DIRS = 16384
D_MODEL = 2048
K = 32
MULTIK = 128

ROWS = 8192
RB_ENC = 512
CB = 2048
NJ = N_DIRS // CB
NI_ENC = ROWS // RB_ENC
RB_DEC = 512
NI_DEC = ROWS // RB_DEC
CBD = 1024
NJD = N_DIRS // CBD


def _enc_kernel(x_ref, wenc_ref, lb_ref, pb_ref, out_ref, s1_ref, s2_ref):
    xc = x_ref[...] - pb_ref[...]
    acc = jax.lax.dot_general(
        xc, wenc_ref[...],
        dimension_numbers=(((1,), (1,)), ((), ())),
        preferred_element_type=jnp.float32,
    )
    acc = acc + lb_ref[...]
    out_ref[...] = acc
    s1_ref[...] = jnp.sum(acc, axis=1)[None, None, :]
    s2_ref[...] = jnp.sum(acc * acc, axis=1)[None, None, :]


def _encoder(x, W_enc, pre_bias, latent_bias):
    grid = (NJ, NI_ENC)
    out_shapes = (
        jax.ShapeDtypeStruct((ROWS, N_DIRS), jnp.float32),
        jax.ShapeDtypeStruct((NJ, 1, ROWS), jnp.float32),
        jax.ShapeDtypeStruct((NJ, 1, ROWS), jnp.float32),
    )
    return pl.pallas_call(
        _enc_kernel,
        grid=grid,
        in_specs=[
            pl.BlockSpec((RB_ENC, D_MODEL), lambda j, i: (i, 0)),
            pl.BlockSpec((CB, D_MODEL), lambda j, i: (j, 0)),
            pl.BlockSpec((1, CB), lambda j, i: (0, j)),
            pl.BlockSpec((1, D_MODEL), lambda j, i: (0, 0)),
        ],
        out_specs=(
            pl.BlockSpec((RB_ENC, CB), lambda j, i: (i, j)),
            pl.BlockSpec((1, 1, RB_ENC), lambda j, i: (j, 0, i)),
            pl.BlockSpec((1, 1, RB_ENC), lambda j, i: (j, 0, i)),
        ),
        out_shape=out_shapes,
    )(x, W_enc, latent_bias.reshape(1, N_DIRS), pre_bias.reshape(1, D_MODEL))


def _dec_kernel(pre_ref, t32_ref, t128_ref, wdec_ref, pb_ref,
                lat_ref, rec_ref, mrec_ref):
    j = pl.program_id(1)
    pre = pre_ref[...]
    relu = jnp.maximum(pre, 0.0)
    lat = jnp.where(pre >= t32_ref[...], relu, 0.0)
    mk = jnp.where(pre >= t128_ref[...], relu, 0.0)
    lat_ref[...] = lat
    w = wdec_ref[...]
    prec = jax.lax.Precision.DEFAULT
    rec = jax.lax.dot_general(
        lat, w, dimension_numbers=(((1,), (1,)), ((), ())),
        preferred_element_type=jnp.float32, precision=prec)
    mrec = jax.lax.dot_general(
        mk, w, dimension_numbers=(((1,), (1,)), ((), ())),
        preferred_element_type=jnp.float32, precision=prec)

    @pl.when(j == 0)
    def _init():
        rec_ref[...] = rec + pb_ref[...]
        mrec_ref[...] = mrec + pb_ref[...]

    @pl.when(j > 0)
    def _acc():
        rec_ref[...] += rec
        mrec_ref[...] += mrec


def _decoder(pre, tau32, tau128, W_dec, pre_bias):
    grid = (NI_DEC, NJD)
    out_shapes = (
        jax.ShapeDtypeStruct((ROWS, N_DIRS), jnp.float32),
        jax.ShapeDtypeStruct((ROWS, D_MODEL), jnp.float32),
        jax.ShapeDtypeStruct((ROWS, D_MODEL), jnp.float32),
    )
    return pl.pallas_call(
        _dec_kernel,
        grid=grid,
        in_specs=[
            pl.BlockSpec((RB_DEC, CBD), lambda i, j: (i, j)),
            pl.BlockSpec((RB_DEC, 1), lambda i, j: (i, 0)),
            pl.BlockSpec((RB_DEC, 1), lambda i, j: (i, 0)),
            pl.BlockSpec((D_MODEL, CBD), lambda i, j: (0, j)),
            pl.BlockSpec((1, D_MODEL), lambda i, j: (0, 0)),
        ],
        out_specs=(
            pl.BlockSpec((RB_DEC, CBD), lambda i, j: (i, j)),
            pl.BlockSpec((RB_DEC, D_MODEL), lambda i, j: (i, 0)),
            pl.BlockSpec((RB_DEC, D_MODEL), lambda i, j: (i, 0)),
        ),
        out_shape=out_shapes,
    )(pre, tau32, tau128, W_dec, pre_bias.reshape(1, D_MODEL))


NC = 2
NS = 16
NW = NC * NS
RPW = ROWS // NW
CAP = 272
NV = N_DIRS // 16
PAD_IDX = 0x7FFFFFFF
NEG_INF = float("-inf")


def _f32_key(v):
    b = lax.bitcast_convert_type(v, jnp.int32)
    flip = jnp.where(b < 0, jnp.int32(-1), jnp.int32(-2147483648))
    return lax.bitcast_convert_type(b ^ flip, jnp.uint32)


def _beats(av, ai, bv, bi):
    return (av > bv) | ((av == bv) & (ai < bi))


def _bitonic_steps():
    k = 2
    while k <= 256:
        d = k // 2
        while d >= 1:
            yield d, k
            d //= 2
        k *= 2


def _sc_select_kernel(pre_hbm, thr_hbm, mkv_hbm, mki_hbm,
                      rowbuf, candv, candi, outv, outi, thrbuf):
    i32 = jnp.int32
    wid = lax.axis_index("s") * NC + lax.axis_index("c")
    base = wid * RPW
    pltpu.sync_copy(thr_hbm.at[pl.ds(base * 16, RPW * 16)], thrbuf)
    lanes = lax.iota(i32, 16)

    def scan_simple(tk):
        def body(b, n):
            v = rowbuf[pl.ds(b * 16, 16)]
            m = _f32_key(v) >= tk
            c = plsc.cumsum(jnp.where(m, i32(1), i32(0)))
            pos = jnp.minimum(n + c - 1, i32(CAP - 1))
            plsc.store_scatter(candv, [pos], v, mask=m)
            plsc.store_scatter(candi, [pos], lanes + b * 16, mask=m)
            return n + plsc.all_reduce_population_count(m)
        return lax.fori_loop(0, NV, body, jnp.zeros((16,), i32))

    def count_pass(tk):
        def body(b, n):
            v = rowbuf[pl.ds(b * 16, 16)]
            return n + plsc.all_reduce_population_count(_f32_key(v) >= tk)
        return lax.fori_loop(0, NV, body, jnp.zeros((16,), i32))

    def scan_careful(tk):
        def body(b, n):
            v = rowbuf[pl.ds(b * 16, 16)]
            kv = _f32_key(v)
            mgt = kv > tk
            meq = kv == tk
            call = plsc.cumsum(jnp.where(mgt | meq, i32(1), i32(0)))
            adm = mgt | (meq & ((n + call - 1) < 128))
            cadm = plsc.cumsum(jnp.where(adm, i32(1), i32(0)))
            pos = jnp.minimum(n + cadm - 1, i32(CAP - 1))
            plsc.store_scatter(candv, [pos], v, mask=adm)
            plsc.store_scatter(candi, [pos], lanes + b * 16, mask=adm)
            return n + plsc.all_reduce_population_count(adm)
        return lax.fori_loop(0, NV, body, jnp.zeros((16,), i32))

    def process_row(rl, carry):
        row = base + rl
        pltpu.sync_copy(pre_hbm.at[row], rowbuf)
        t0k = _f32_key(thrbuf[pl.ds(rl * 16, 16)])
        n0 = scan_simple(t0k)
        n0s = jnp.max(n0)

        def fixup(_):
            big = jnp.full((16,), jnp.uint32(0xFFFFFFFF))
            zero = jnp.zeros((16,), jnp.uint32)
            over = n0s > 256
            lo0 = jnp.where(over, t0k, zero)
            hi0 = jnp.where(over, big, t0k)

            def bcond(c):
                lo, hi = c
                return jnp.max(hi - lo) > 1

            def bbody(c):
                lo, hi = c
                mid = lo + ((hi - lo) >> 1)
                ge = jnp.max(count_pass(mid)) >= 128
                return (jnp.where(ge, mid, lo), jnp.where(ge, hi, mid))

            lo, _hi = lax.while_loop(bcond, bbody, (lo0, hi0))
            return scan_careful(lo)

        ok = (n0s >= 128) & (n0s <= 256)
        nfin = lax.cond(ok, lambda _: n0, fixup, 0)

        for r in range(17):
            pvec = lanes + r * 16
            mpad = pvec >= nfin
            vv = candv[pl.ds(r * 16, 16)]
            iv = candi[pl.ds(r * 16, 16)]
            candv[pl.ds(r * 16, 16)] = jnp.where(mpad, NEG_INF, vv)
            candi[pl.ds(r * 16, 16)] = jnp.where(mpad, PAD_IDX, iv)

        V = [candv[pl.ds(r * 16, 16)] for r in range(16)]
        I = [candi[pl.ds(r * 16, 16)] for r in range(16)]
        for d, k in _bitonic_steps():
            if d < 16:
                for ra in range(16):
                    if ra & d:
                        continue
                    rb = ra | d
                    keep = ((lanes * 16 + ra) & k) == 0
                    beats = _beats(V[ra], I[ra], V[rb], I[rb])
                    sel = beats == keep
                    nva = jnp.where(sel, V[ra], V[rb])
                    nvb = jnp.where(sel, V[rb], V[ra])
                    nia = jnp.where(sel, I[ra], I[rb])
                    nib = jnp.where(sel, I[rb], I[ra])
                    V[ra], V[rb], I[ra], I[rb] = nva, nvb, nia, nib
            else:
                dl = d // 16
                perm = lanes ^ dl
                amq = (lanes & dl) != 0
                for r in range(16):
                    keep = (((lanes * 16 + r) & k) == 0) ^ amq
                    vp = jnp.take_along_axis(V[r], perm, axis=0,
                                             mode="promise_in_bounds")
                    ip = jnp.take_along_axis(I[r], perm, axis=0,
                                             mode="promise_in_bounds")
                    beats = _beats(V[r], I[r], vp, ip)
                    sel = beats == keep
                    V[r] = jnp.where(sel, V[r], vp)
                    I[r] = jnp.where(sel, I[r], ip)

        halfm = lanes < 8
        for r in range(16):
            tgt = lanes * 16 + r
            plsc.store_scatter(outv, [tgt], V[r], mask=halfm)
            plsc.store_scatter(outi, [tgt], I[r], mask=halfm)
        pltpu.sync_copy(outv, mkv_hbm.at[row])
        pltpu.sync_copy(outi, mki_hbm.at[row])
        return carry

    lax.fori_loop(0, RPW, process_row, 0)


def _sc_select(pre, t0):
    thr_flat = jnp.tile(t0[:, None], (1, 16)).reshape(-1)
    mesh = plsc.VectorSubcoreMesh(core_axis_name="c", subcore_axis_name="s")
    f = pl.kernel(
        _sc_select_kernel,
        out_type=(
            jax.ShapeDtypeStruct((ROWS, MULTIK), jnp.float32),
            jax.ShapeDtypeStruct((ROWS, MULTIK), jnp.int32),
        ),
        mesh=mesh,
        compiler_params=pltpu.CompilerParams(needs_layout_passes=False),
        scratch_types=[
            pltpu.VMEM((N_DIRS,), jnp.float32),
            pltpu.VMEM((CAP,), jnp.float32),
            pltpu.VMEM((CAP,), jnp.int32),
            pltpu.VMEM((MULTIK,), jnp.float32),
            pltpu.VMEM((MULTIK,), jnp.int32),
            pltpu.VMEM((RPW * 16,), jnp.float32),
        ],
    )
    return f(pre, thr_flat)


def kernel(x, W_enc, W_dec, pre_bias, latent_bias):
    pre, s1, s2 = _encoder(x, W_enc, pre_bias, latent_bias)

    s1t = jnp.sum(s1, axis=0).reshape(ROWS)
    s2t = jnp.sum(s2, axis=0).reshape(ROWS)
    mu = s1t / N_DIRS
    sigma = jnp.sqrt(jnp.maximum(s2t / N_DIRS - mu * mu, 0.0))
    t0 = mu + 2.30 * sigma

    mkv_raw, multik_indices = _sc_select(pre, t0)
    topk_indices = multik_indices[:, :K]
    tau32 = mkv_raw[:, K - 1:K]
    tau128 = mkv_raw[:, MULTIK - 1:MULTIK]

    multik_values = jnp.maximum(mkv_raw, 0.0)
    topk_values = multik_values[:, :K]

    latents, recons, multik_recons = _decoder(pre, tau32, tau128, W_dec, pre_bias)
    return (recons, topk_indices, topk_values, multik_indices, multik_values,
            multik_recons, pre, latents)
